# baseline (speedup 1.0000x reference)
import jax
import jax.numpy as jnp
from jax.experimental import pallas as pl
from jax.experimental.pallas import tpu as pltpu


def _sel_kernel(mask_ref, sel_ref, nsel_ref):
    c = mask_ref.shape[-1]
    nz = mask_ref[...] != 0.0
    nzi = nz.astype(jnp.int32)
    row = jax.lax.broadcasted_iota(jnp.int32, (c, c), 0)
    col = jax.lax.broadcasted_iota(jnp.int32, (c, c), 1)
    rank = jnp.sum((nz & (col < row)).astype(jnp.int32), axis=1)
    m = nz & (jnp.broadcast_to(rank[None, :], (c, c)) == row)
    sel = jnp.sum(jnp.where(m, col, 0), axis=1)
    sel_ref[...] = sel.reshape(1, c)
    nsel_ref[...] = jnp.sum(nzi, axis=-1, keepdims=True)


def _copy_kernel(sel_ref, nsel_ref, in_ref, out_ref):
    del sel_ref
    j = pl.program_id(0)

    @pl.when(j < nsel_ref[0])
    def _valid():
        out_ref[...] = in_ref[...]

    @pl.when(j >= nsel_ref[0])
    def _invalid():
        out_ref[...] = jnp.full_like(out_ref, jnp.nan)


def kernel(input_tensor, indexes):
    n, c, h0, w0 = input_tensor.shape
    hw = h0 * w0
    w = 128
    h = hw // 128
    if h * w == hw:
        input_tensor = input_tensor.reshape(n, c, h, w)
    else:
        h, w = h0, w0

    sel, nsel = pl.pallas_call(
        _sel_kernel,
        out_shape=(
            jax.ShapeDtypeStruct((1, c), jnp.int32),
            jax.ShapeDtypeStruct((1, 1), jnp.int32),
        ),
    )(indexes.reshape(1, c))
    sel = sel.reshape(c)
    nsel = nsel.reshape(1)

    grid_spec = pltpu.PrefetchScalarGridSpec(
        num_scalar_prefetch=2,
        grid=(c,),
        in_specs=[
            pl.BlockSpec(
                (n, 1, h, w), lambda j, sel_ref, nsel_ref: (0, sel_ref[j], 0, 0)
            )
        ],
        out_specs=pl.BlockSpec(
            (n, 1, h, w), lambda j, sel_ref, nsel_ref: (0, j, 0, 0)
        ),
    )
    out = pl.pallas_call(
        _copy_kernel,
        grid_spec=grid_spec,
        out_shape=jax.ShapeDtypeStruct((n, c, h, w), input_tensor.dtype),
        compiler_params=pltpu.CompilerParams(
            dimension_semantics=("parallel",),
        ),
    )(sel, nsel, input_tensor)
    return out.reshape(n, c, h0, w0)


# speedup vs baseline: 4.4294x; 4.4294x over previous
import jax
import jax.numpy as jnp
from jax.experimental import pallas as pl
from jax.experimental.pallas import tpu as pltpu


def _sel_kernel(mask_ref, sel_ref, nsel_ref):
    c = mask_ref.shape[-1]
    nz = mask_ref[...] != 0.0
    nzi = nz.astype(jnp.int32)
    row = jax.lax.broadcasted_iota(jnp.int32, (c, c), 0)
    col = jax.lax.broadcasted_iota(jnp.int32, (c, c), 1)
    rank = jnp.sum((nz & (col < row)).astype(jnp.int32), axis=1)
    m = nz & (jnp.broadcast_to(rank[None, :], (c, c)) == row)
    sel = jnp.sum(jnp.where(m, col, 0), axis=1)
    sel_ref[...] = sel.reshape(1, c)
    nsel_ref[...] = jnp.sum(nzi, axis=-1, keepdims=True)


def _copy_kernel(sel_ref, nsel_ref, in_a, in_b, out_ref):
    del sel_ref
    k = pl.program_id(0)
    nsel = nsel_ref[0]

    @pl.when(2 * k < nsel)
    def _valid_a():
        out_ref[:, 0:1] = in_a[...]

    @pl.when(2 * k >= nsel)
    def _invalid_a():
        out_ref[:, 0:1] = jnp.full_like(in_a, jnp.nan)

    @pl.when(2 * k + 1 < nsel)
    def _valid_b():
        out_ref[:, 1:2] = in_b[...]

    @pl.when(2 * k + 1 >= nsel)
    def _invalid_b():
        out_ref[:, 1:2] = jnp.full_like(in_b, jnp.nan)


def kernel(input_tensor, indexes):
    n, c, h, w = input_tensor.shape

    sel, nsel = pl.pallas_call(
        _sel_kernel,
        out_shape=(
            jax.ShapeDtypeStruct((1, c), jnp.int32),
            jax.ShapeDtypeStruct((1, 1), jnp.int32),
        ),
    )(indexes.reshape(1, c))
    sel = sel.reshape(c)
    nsel = nsel.reshape(1)

    grid_spec = pltpu.PrefetchScalarGridSpec(
        num_scalar_prefetch=2,
        grid=(c // 2,),
        in_specs=[
            pl.BlockSpec(
                (n, 1, h, w),
                lambda k, sel_ref, nsel_ref: (0, sel_ref[2 * k], 0, 0),
            ),
            pl.BlockSpec(
                (n, 1, h, w),
                lambda k, sel_ref, nsel_ref: (0, sel_ref[2 * k + 1], 0, 0),
            ),
        ],
        out_specs=pl.BlockSpec(
            (n, 2, h, w), lambda k, sel_ref, nsel_ref: (0, k, 0, 0)
        ),
    )
    return pl.pallas_call(
        _copy_kernel,
        grid_spec=grid_spec,
        out_shape=jax.ShapeDtypeStruct((n, c, h, w), input_tensor.dtype),
        compiler_params=pltpu.CompilerParams(
            dimension_semantics=("parallel",),
        ),
    )(sel, nsel, input_tensor, input_tensor)


# speedup vs baseline: 4.5460x; 1.0263x over previous
import jax
import jax.numpy as jnp
from jax.experimental import pallas as pl
from jax.experimental.pallas import tpu as pltpu


def _sel_kernel(mask_ref, sel_ref, nsel_ref):
    c = mask_ref.shape[-1]
    nz = mask_ref[...] != 0.0
    nzi = nz.astype(jnp.int32)
    row = jax.lax.broadcasted_iota(jnp.int32, (c, c), 0)
    col = jax.lax.broadcasted_iota(jnp.int32, (c, c), 1)
    rank = jnp.sum((nz & (col < row)).astype(jnp.int32), axis=1)
    m = nz & (jnp.broadcast_to(rank[None, :], (c, c)) == row)
    sel = jnp.sum(jnp.where(m, col, 0), axis=1)
    sel_ref[...] = sel.reshape(1, c)
    nsel_ref[...] = jnp.sum(nzi, axis=-1, keepdims=True)


_U = 4


def _copy_kernel(sel_ref, nsel_ref, *refs):
    del sel_ref
    ins = refs[:_U]
    out_ref = refs[_U]
    k = pl.program_id(0)
    nsel = nsel_ref[0]
    for u in range(_U):
        j = _U * k + u

        @pl.when(j < nsel)
        def _valid(u=u):
            out_ref[:, u : u + 1] = ins[u][...]

        @pl.when(j >= nsel)
        def _invalid(u=u):
            out_ref[:, u : u + 1] = jnp.full_like(ins[u], jnp.nan)


def kernel(input_tensor, indexes):
    n, c, h, w = input_tensor.shape

    sel, nsel = pl.pallas_call(
        _sel_kernel,
        out_shape=(
            jax.ShapeDtypeStruct((1, c), jnp.int32),
            jax.ShapeDtypeStruct((1, 1), jnp.int32),
        ),
    )(indexes.reshape(1, c))
    sel = sel.reshape(c)
    nsel = nsel.reshape(1)

    def _in_spec(u):
        return pl.BlockSpec(
            (n, 1, h, w),
            lambda k, sel_ref, nsel_ref: (0, sel_ref[_U * k + u], 0, 0),
        )

    grid_spec = pltpu.PrefetchScalarGridSpec(
        num_scalar_prefetch=2,
        grid=(c // _U,),
        in_specs=[_in_spec(u) for u in range(_U)],
        out_specs=pl.BlockSpec(
            (n, _U, h, w), lambda k, sel_ref, nsel_ref: (0, k, 0, 0)
        ),
    )
    return pl.pallas_call(
        _copy_kernel,
        grid_spec=grid_spec,
        out_shape=jax.ShapeDtypeStruct((n, c, h, w), input_tensor.dtype),
        compiler_params=pltpu.CompilerParams(
            dimension_semantics=("parallel",),
        ),
    )(sel, nsel, *([input_tensor] * _U))
